# baseline (speedup 1.0000x reference)
import jax
import jax.numpy as jnp
from jax.experimental import pallas as pl

N_ROWS = 4096
N_COLS = 1000
BLOCK_ROWS = 512


def _cov_kernel(p_ref, t_ref, out_ref):
    p = p_ref[...]
    t = t_ref[...]
    masked = jnp.where(t > 0, p, jnp.inf)
    rowmin = jnp.min(masked, axis=1, keepdims=True)
    cov = jnp.sum((p >= rowmin).astype(jnp.float32), axis=1)
    cov = jnp.where(jnp.isfinite(rowmin[:, 0]), cov, 0.0)
    total = jnp.sum(cov)

    @pl.when(pl.program_id(0) == 0)
    def _():
        out_ref[...] = jnp.zeros((1, 1), jnp.float32)

    out_ref[...] += total[None, None]


def kernel(predict_probs, true_labels):
    grid = (N_ROWS // BLOCK_ROWS,)
    out = pl.pallas_call(
        _cov_kernel,
        grid=grid,
        in_specs=[
            pl.BlockSpec((BLOCK_ROWS, N_COLS), lambda i: (i, 0)),
            pl.BlockSpec((BLOCK_ROWS, N_COLS), lambda i: (i, 0)),
        ],
        out_specs=pl.BlockSpec((1, 1), lambda i: (0, 0)),
        out_shape=jax.ShapeDtypeStruct((1, 1), jnp.float32),
    )(predict_probs, true_labels)
    return out[0, 0] / N_ROWS


# speedup vs baseline: 1.0427x; 1.0427x over previous
import jax
import jax.numpy as jnp
from jax.experimental import pallas as pl

N_ROWS = 4096
N_COLS = 1000
BLOCK_ROWS = 1024


def _cov_kernel(p_ref, t_ref, out_ref):
    p = p_ref[...]
    t = t_ref[...]
    masked = jnp.where(t > 0, p, jnp.inf)
    rowmin = jnp.min(masked, axis=1, keepdims=True)
    cov = jnp.sum((p >= rowmin).astype(jnp.float32), axis=1)
    cov = jnp.where(jnp.isfinite(rowmin[:, 0]), cov, 0.0)
    total = jnp.sum(cov)

    @pl.when(pl.program_id(0) == 0)
    def _():
        out_ref[...] = jnp.zeros((1, 1), jnp.float32)

    out_ref[...] += total[None, None]


def kernel(predict_probs, true_labels):
    grid = (N_ROWS // BLOCK_ROWS,)
    out = pl.pallas_call(
        _cov_kernel,
        grid=grid,
        in_specs=[
            pl.BlockSpec((BLOCK_ROWS, N_COLS), lambda i: (i, 0)),
            pl.BlockSpec((BLOCK_ROWS, N_COLS), lambda i: (i, 0)),
        ],
        out_specs=pl.BlockSpec((1, 1), lambda i: (0, 0)),
        out_shape=jax.ShapeDtypeStruct((1, 1), jnp.float32),
    )(predict_probs, true_labels)
    return out[0, 0] / N_ROWS
